# baseline (speedup 1.0000x reference)
import functools

import jax
import jax.numpy as jnp
from jax import lax
from jax.experimental import pallas as pl
from jax.experimental.pallas import tpu as pltpu
from jax.experimental.pallas import tpu_sc as plsc

_NC = 2
_NS = 16
_NW = _NC * _NS
_BI = 128
_SP = 56
_S = 50
_D = 64


def _make_gather():
    mesh = plsc.VectorSubcoreMesh(core_axis_name="c", subcore_axis_name="s")

    @functools.partial(
        pl.kernel,
        mesh=mesh,
        out_type=jax.ShapeDtypeStruct((_S, _D // 8, _NW, 8, _BI), jnp.float32),
        scratch_types=[
            pltpu.VMEM((_SP, _BI), jnp.int32),
            pltpu.VMEM((_BI, _D), jnp.float32),
            pltpu.VMEM((_BI, _D), jnp.float32),
            pltpu.VMEM((_D, _BI), jnp.float32),
            pltpu.VMEM((_D, _BI), jnp.float32),
            pltpu.SemaphoreType.DMA,
            pltpu.SemaphoreType.DMA,
            pltpu.SemaphoreType.DMA,
            pltpu.SemaphoreType.DMA,
        ],
        compiler_params=pltpu.CompilerParams(
            use_tc_tiling_on_sc=False, needs_layout_passes=False
        ),
    )
    def k(idx_hbm, table_hbm, out_hbm, idx_v, r0, r1, t0, t1, sg0, sg1, sw0, sw1):
        wid = lax.axis_index("s") * _NC + lax.axis_index("c")
        rows = (r0, r1)
        rowsT = (t0, t1)
        sg = (sg0, sg1)
        sw = (sw0, sw1)

        for sb in range(_SP // 8):
            pltpu.sync_copy(idx_hbm.at[sb, wid], idx_v.at[pl.ds(sb * 8, 8)])

        def g_start(s, b):
            pltpu.make_async_copy(
                table_hbm.at[idx_v.at[s]], rows[b], sg[b]
            ).start()

        def g_wait(b):
            pltpu.make_async_copy(
                table_hbm.at[idx_v.at[0]], rows[b], sg[b]
            ).wait()

        def w_start(s, b):
            for fb in range(_D // 8):
                pltpu.make_async_copy(
                    rowsT[b].at[pl.ds(fb * 8, 8)], out_hbm.at[s, fb, wid], sw[b]
                ).start()

        def w_wait(b):
            for fb in range(_D // 8):
                pltpu.make_async_copy(
                    rowsT[b].at[pl.ds(fb * 8, 8)], out_hbm.at[0, fb, wid], sw[b]
                ).wait()

        lanes = lax.iota(jnp.int32, 16)
        rowvecs = [lanes + g * 16 for g in range(8)]

        def transpose(b):
            def body(f, carry):
                col = jnp.broadcast_to(f, (16,)).astype(jnp.int32)
                for g in range(8):
                    v = plsc.load_gather(rows[b], [rowvecs[g], col])
                    rowsT[b][f, pl.ds(g * 16, 16)] = v
                return carry

            lax.fori_loop(0, _D, body, 0)

        def unit(s, b, first, last):
            g_wait(b)
            if not last:
                g_start(s + 1, 1 - b)
            if not first:
                w_wait(b)
            transpose(b)
            w_start(s, b)

        g_start(0, 0)
        unit(0, 0, True, False)
        unit(1, 1, True, False)

        def pair(p, carry):
            unit(2 * p, 0, False, False)
            unit(2 * p + 1, 1, False, False)
            return carry

        lax.fori_loop(1, _S // 2 - 1, pair, 0)

        unit(_S - 2, 0, False, False)
        unit(_S - 1, 1, False, True)

        w_wait(0)
        w_wait(1)

    return k


def kernel(input, weight):
    idx = input.astype(jnp.int32)
    idxp = jnp.pad(idx, ((0, 0), (0, _SP - _S)))
    idx4 = (
        idxp.T.reshape(_SP // 8, 8, _NW, _BI).transpose(0, 2, 1, 3)
    )
    out5 = _make_gather()(idx4, weight)
    return out5.transpose(2, 4, 0, 1, 3).reshape(4096, _S, _D)


# speedup vs baseline: 1.5120x; 1.5120x over previous
import functools

import jax
import jax.numpy as jnp
from jax import lax
from jax.experimental import pallas as pl
from jax.experimental.pallas import tpu as pltpu
from jax.experimental.pallas import tpu_sc as plsc

_NC = 2
_NS = 16
_NW = _NC * _NS
_BI = 128
_SP = 56
_S = 50
_D = 64


def _make_gather():
    mesh = plsc.VectorSubcoreMesh(core_axis_name="c", subcore_axis_name="s")

    @functools.partial(
        pl.kernel,
        mesh=mesh,
        out_type=jax.ShapeDtypeStruct((_S, _D // 8, _NW, 8, _BI), jnp.float32),
        scratch_types=[
            pltpu.VMEM((_SP, _BI), jnp.int32),
            pltpu.VMEM((_BI, _D), jnp.float32),
            pltpu.VMEM((_BI, _D), jnp.float32),
            pltpu.VMEM((_D, _BI), jnp.float32),
            pltpu.VMEM((_D, _BI), jnp.float32),
            pltpu.SemaphoreType.DMA,
            pltpu.SemaphoreType.DMA,
            pltpu.SemaphoreType.DMA,
            pltpu.SemaphoreType.DMA,
        ],
        compiler_params=pltpu.CompilerParams(
            use_tc_tiling_on_sc=False, needs_layout_passes=False
        ),
    )
    def k(idx_hbm, table_hbm, out_hbm, idx_v, r0, r1, t0, t1, sg0, sg1, sw0, sw1):
        wid = lax.axis_index("s") * _NC + lax.axis_index("c")
        rows = (r0, r1)
        rowsT = (t0, t1)
        sg = (sg0, sg1)
        sw = (sw0, sw1)

        for sb in range(_SP // 8):
            pltpu.sync_copy(idx_hbm.at[sb, wid], idx_v.at[pl.ds(sb * 8, 8)])

        def g_start(s, b):
            pltpu.make_async_copy(
                table_hbm.at[idx_v.at[s]], rows[b], sg[b]
            ).start()

        def g_wait(b):
            pltpu.make_async_copy(
                table_hbm.at[idx_v.at[0]], rows[b], sg[b]
            ).wait()

        def w_start(s, b):
            for fb in range(_D // 8):
                pltpu.make_async_copy(
                    rowsT[b].at[pl.ds(fb * 8, 8)], out_hbm.at[s, fb, wid], sw[b]
                ).start()

        def w_wait(b):
            for fb in range(_D // 8):
                pltpu.make_async_copy(
                    rowsT[b].at[pl.ds(fb * 8, 8)], out_hbm.at[0, fb, wid], sw[b]
                ).wait()

        lanes = lax.iota(jnp.int32, 16)
        rowvecs = [lanes + g * 16 for g in range(8)]

        def transpose(b):
            @plsc.parallel_loop(0, _D, 1, unroll=8)
            def body(f):
                col = jnp.broadcast_to(f, (16,)).astype(jnp.int32)
                for g in range(8):
                    v = plsc.load_gather(rows[b], [rowvecs[g], col])
                    rowsT[b][f, pl.ds(g * 16, 16)] = v

        def unit(s, b, first, last):
            g_wait(b)
            if not last:
                g_start(s + 1, 1 - b)
            if not first:
                w_wait(b)
            transpose(b)
            w_start(s, b)

        g_start(0, 0)
        unit(0, 0, True, False)
        unit(1, 1, True, False)

        def pair(p, carry):
            unit(2 * p, 0, False, False)
            unit(2 * p + 1, 1, False, False)
            return carry

        lax.fori_loop(1, _S // 2 - 1, pair, 0)

        unit(_S - 2, 0, False, False)
        unit(_S - 1, 1, False, True)

        w_wait(0)
        w_wait(1)

    return k


def kernel(input, weight):
    idx = input.astype(jnp.int32)
    idxp = jnp.pad(idx, ((0, 0), (0, _SP - _S)))
    idx4 = (
        idxp.T.reshape(_SP // 8, 8, _NW, _BI).transpose(0, 2, 1, 3)
    )
    out5 = _make_gather()(idx4, weight)
    return out5.transpose(2, 4, 0, 1, 3).reshape(4096, _S, _D)


# speedup vs baseline: 2.7207x; 1.7994x over previous
import functools

import jax
import jax.numpy as jnp
from jax import lax
from jax.experimental import pallas as pl
from jax.experimental.pallas import tpu as pltpu
from jax.experimental.pallas import tpu_sc as plsc

_NC = 2
_NS = 16
_NW = _NC * _NS
_BI = 128
_SP = 56
_S = 50
_D = 64


def _make_gather():
    mesh = plsc.VectorSubcoreMesh(core_axis_name="c", subcore_axis_name="s")

    @functools.partial(
        pl.kernel,
        mesh=mesh,
        out_type=jax.ShapeDtypeStruct((_S, _D // 8, _NW, 8, _BI), jnp.float32),
        scratch_types=[
            pltpu.VMEM((_SP, _BI), jnp.int32),
            pltpu.VMEM((_BI, _D), jnp.float32),
            pltpu.VMEM((_BI, _D), jnp.float32),
            pltpu.VMEM((_D, _BI + 1), jnp.float32),
            pltpu.VMEM((_D, _BI + 1), jnp.float32),
            pltpu.SemaphoreType.DMA,
            pltpu.SemaphoreType.DMA,
            pltpu.SemaphoreType.DMA,
            pltpu.SemaphoreType.DMA,
        ],
        compiler_params=pltpu.CompilerParams(
            use_tc_tiling_on_sc=False, needs_layout_passes=False
        ),
    )
    def k(idx_hbm, table_hbm, out_hbm, idx_v, r0, r1, t0, t1, sg0, sg1, sw0, sw1):
        wid = lax.axis_index("s") * _NC + lax.axis_index("c")
        rows = (r0, r1)
        rowsT = (t0, t1)
        sg = (sg0, sg1)
        sw = (sw0, sw1)

        for sb in range(_SP // 8):
            pltpu.sync_copy(idx_hbm.at[sb, wid], idx_v.at[pl.ds(sb * 8, 8)])

        def g_start(s, b):
            pltpu.make_async_copy(
                table_hbm.at[idx_v.at[s]], rows[b], sg[b]
            ).start()

        def g_wait(b):
            pltpu.make_async_copy(
                table_hbm.at[idx_v.at[0]], rows[b], sg[b]
            ).wait()

        def w_start(s, b):
            for fb in range(_D // 8):
                pltpu.make_async_copy(
                    rowsT[b].at[pl.ds(fb * 8, 8), pl.ds(0, _BI)],
                    out_hbm.at[s, fb, wid],
                    sw[b],
                ).start()

        def w_wait(b):
            for fb in range(_D // 8):
                pltpu.make_async_copy(
                    rowsT[b].at[pl.ds(fb * 8, 8), pl.ds(0, _BI)],
                    out_hbm.at[0, fb, wid],
                    sw[b],
                ).wait()

        lanes = lax.iota(jnp.int32, 16)
        fvecs = [lanes + g * 16 for g in range(_D // 16)]

        def transpose(b):
            @plsc.parallel_loop(0, _BI, 1, unroll=8)
            def body(t):
                col = jnp.broadcast_to(t, (16,)).astype(jnp.int32)
                for g in range(_D // 16):
                    v = rows[b][t, pl.ds(g * 16, 16)]
                    plsc.store_scatter(rowsT[b], [fvecs[g], col], v)

        def unit(s, b, first, last):
            g_wait(b)
            if not last:
                g_start(s + 1, 1 - b)
            if not first:
                w_wait(b)
            transpose(b)
            w_start(s, b)

        g_start(0, 0)
        unit(0, 0, True, False)
        unit(1, 1, True, False)

        def pair(p, carry):
            unit(2 * p, 0, False, False)
            unit(2 * p + 1, 1, False, False)
            return carry

        lax.fori_loop(1, _S // 2 - 1, pair, 0)

        unit(_S - 2, 0, False, False)
        unit(_S - 1, 1, False, True)

        w_wait(0)
        w_wait(1)

    return k


def kernel(input, weight):
    idx = input.astype(jnp.int32)
    idxp = jnp.pad(idx, ((0, 0), (0, _SP - _S)))
    idx4 = (
        idxp.T.reshape(_SP // 8, 8, _NW, _BI).transpose(0, 2, 1, 3)
    )
    out5 = _make_gather()(idx4, weight)
    return out5.transpose(2, 4, 0, 1, 3).reshape(4096, _S, _D)


# speedup vs baseline: 2.9498x; 1.0842x over previous
import functools

import jax
import jax.numpy as jnp
from jax import lax
from jax.experimental import pallas as pl
from jax.experimental.pallas import tpu as pltpu
from jax.experimental.pallas import tpu_sc as plsc

_NC = 2
_NS = 16
_NW = _NC * _NS
_BI = 128
_SP = 56
_S = 50
_D = 64
_NB = 4


def _make_gather():
    mesh = plsc.VectorSubcoreMesh(core_axis_name="c", subcore_axis_name="s")

    @functools.partial(
        pl.kernel,
        mesh=mesh,
        out_type=jax.ShapeDtypeStruct((_S, _D // 8, _NW, 8, _BI), jnp.float32),
        scratch_types=[
            pltpu.VMEM((_SP, _BI), jnp.int32),
            pltpu.VMEM((_NB, _BI, _D), jnp.float32),
            pltpu.VMEM((_NB, _D, _BI + 1), jnp.float32),
            [pltpu.SemaphoreType.DMA] * _NB,
            [pltpu.SemaphoreType.DMA] * _NB,
        ],
        compiler_params=pltpu.CompilerParams(
            use_tc_tiling_on_sc=False, needs_layout_passes=False
        ),
    )
    def k(idx_hbm, table_hbm, out_hbm, idx_v, rows_v, rowsT_v, sg, sw):
        wid = lax.axis_index("s") * _NC + lax.axis_index("c")

        for sb in range(_SP // 8):
            pltpu.sync_copy(idx_hbm.at[sb, wid], idx_v.at[pl.ds(sb * 8, 8)])

        def g_start(s, b):
            pltpu.make_async_copy(
                table_hbm.at[idx_v.at[s]], rows_v.at[b], sg[b]
            ).start()

        def g_wait(b):
            pltpu.make_async_copy(
                table_hbm.at[idx_v.at[0]], rows_v.at[b], sg[b]
            ).wait()

        def w_start(s, b):
            for fb in range(_D // 8):
                pltpu.make_async_copy(
                    rowsT_v.at[b, pl.ds(fb * 8, 8), pl.ds(0, _BI)],
                    out_hbm.at[s, fb, wid],
                    sw[b],
                ).start()

        def w_wait(b):
            for fb in range(_D // 8):
                pltpu.make_async_copy(
                    rowsT_v.at[b, pl.ds(fb * 8, 8), pl.ds(0, _BI)],
                    out_hbm.at[0, fb, wid],
                    sw[b],
                ).wait()

        lanes = lax.iota(jnp.int32, 16)
        fvecs = [lanes + g * 16 for g in range(_D // 16)]

        def transpose(b):
            @plsc.parallel_loop(0, _BI, 1, unroll=16)
            def body(t):
                col = jnp.broadcast_to(t, (16,)).astype(jnp.int32)
                for g in range(_D // 16):
                    v = rows_v[b, t, pl.ds(g * 16, 16)]
                    plsc.store_scatter(rowsT_v.at[b], [fvecs[g], col], v)

        def unit(s, b, first, fire):
            g_wait(b)
            if fire:
                g_start(s + (_NB - 1), (b + _NB - 1) % _NB)
            if not first:
                w_wait(b)
            transpose(b)
            w_start(s, b)

        for b in range(_NB - 1):
            g_start(b, b)
        for s in range(_NB):
            unit(s, s % _NB, True, s + _NB - 1 <= _S - 1)

        def quad(p, carry):
            for q in range(_NB):
                unit(_NB * p + q, q, False, True)
            return carry

        lax.fori_loop(1, (_S - _NB - 2) // _NB, quad, 0)

        for s in range(44, _S):
            unit(s, s % _NB, False, s + _NB - 1 <= _S - 1)

        for b in range(_NB):
            w_wait(b)

    return k


def kernel(input, weight):
    idx = input.astype(jnp.int32)
    idxp = jnp.pad(idx, ((0, 0), (0, _SP - _S)))
    idx4 = (
        idxp.T.reshape(_SP // 8, 8, _NW, _BI).transpose(0, 2, 1, 3)
    )
    out5 = _make_gather()(idx4, weight)
    return out5.transpose(2, 4, 0, 1, 3).reshape(4096, _S, _D)
